# baseline (speedup 1.0000x reference)
import functools
import jax
import jax.numpy as jnp
from jax import lax
from jax.experimental import pallas as pl
from jax.experimental.pallas import tpu as pltpu, tpu_sc as plsc

D = 256
N = 8192
NC = 2
NS = 16
NW = NC * NS
RPW = N // NW
L = 16

_MESH = plsc.VectorSubcoreMesh(
    core_axis_name="c", subcore_axis_name="s", num_cores=NC, num_subcores=NS)
_PARAMS = pltpu.CompilerParams(needs_layout_passes=False)


def _stage1(cb_flat, z_hbm, out_d, out_i, cb_v, z_v, res_v):
    c = lax.axis_index("c")
    s = lax.axis_index("s")
    wid = c * NS + s
    base_row = wid * RPW

    pltpu.sync_copy(cb_flat.at[pl.ds(wid * (RPW * D), RPW * D)], cb_v)
    pltpu.sync_copy(z_hbm, z_v)

    z_vecs = [z_v[pl.ds(L * d, L)] for d in range(D // L)]

    R = 8

    def group_body(g, carry):
        best, bidx = carry
        base = g * (R * D)
        sums = []
        for j in range(R):
            acc = jnp.zeros((L,), jnp.float32)
            for d in range(D // L):
                t = cb_v[pl.ds(base + j * D + L * d, L)] - z_vecs[d]
                acc = acc + t * t
            sums.append(jnp.sum(acc))
        for j in range(R):
            r = g * R + j
            m = sums[j] < best
            best = jnp.where(m, sums[j], best)
            bidx = jnp.where(m, base_row + r, bidx)
        return best, bidx

    init = (jnp.float32(jnp.inf), jnp.int32(0))
    best, bidx = lax.fori_loop(0, RPW // R, group_body, init)

    res_v[pl.ds(0, L)] = jnp.broadcast_to(best, (L,))
    res_v[pl.ds(L, L)] = plsc.bitcast(
        jnp.broadcast_to(bidx, (L,)).astype(jnp.int32), jnp.float32)
    pltpu.sync_copy(res_v.at[pl.ds(0, L)], out_d.at[pl.ds(wid * L, L)])
    pltpu.sync_copy(res_v.at[pl.ds(L, L)], out_i.at[pl.ds(wid * L, L)])


def _stage2(cb2d, out_d, out_i, row_out, idx_out, dist_out,
            d_v, i_v, iv_v, row_v, o_v, sem):
    c = lax.axis_index("c")
    s = lax.axis_index("s")
    wid = c * NS + s

    @pl.when(wid == 0)
    def _():
        pltpu.sync_copy(out_d, d_v)
        pltpu.sync_copy(out_i, i_v)
        best = jnp.full((L,), jnp.inf, jnp.float32)
        bidx = jnp.zeros((L,), jnp.int32)
        for w in range(NW):
            dw = d_v[pl.ds(w * L, L)]
            iw = plsc.bitcast(i_v[pl.ds(w * L, L)], jnp.int32)
            m = dw < best
            best = jnp.where(m, dw, best)
            bidx = jnp.where(m, iw, bidx)
        iv_v[...] = bidx
        pltpu.async_copy(cb2d.at[iv_v.at[pl.ds(0, 1)]], row_v, sem).wait()
        for d in range(D // L):
            o_v[pl.ds(L * d, L)] = row_v[0, pl.ds(L * d, L)]
        pltpu.sync_copy(o_v, row_out)
        pltpu.sync_copy(iv_v, idx_out)
        bi = plsc.bitcast(best, jnp.int32)
        g = plsc.bitcast(
            jnp.int32(0x1FBD1DF5) + lax.shift_right_logical(bi, 1),
            jnp.float32)
        half = jnp.float32(0.5)
        for _ in range(4):
            g = half * (g + best / g)
        g = jnp.where(best == 0.0, jnp.zeros((L,), jnp.float32), g)
        d_v[pl.ds(0, L)] = g
        pltpu.sync_copy(d_v.at[pl.ds(0, L)], dist_out)


@jax.jit
def kernel(z, codebook):
    zf = z.reshape(D)
    cb_flat = codebook.reshape(N * D)

    out_d, out_i = pl.kernel(
        _stage1,
        out_type=(
            jax.ShapeDtypeStruct((NW * L,), jnp.float32),
            jax.ShapeDtypeStruct((NW * L,), jnp.float32),
        ),
        mesh=_MESH,
        compiler_params=_PARAMS,
        scratch_types=[
            pltpu.VMEM((RPW * D,), jnp.float32),
            pltpu.VMEM((D,), jnp.float32),
            pltpu.VMEM((2 * L,), jnp.float32),
        ],
    )(cb_flat, zf)

    row, idx, dist = pl.kernel(
        _stage2,
        out_type=(
            jax.ShapeDtypeStruct((D,), jnp.float32),
            jax.ShapeDtypeStruct((L,), jnp.int32),
            jax.ShapeDtypeStruct((L,), jnp.float32),
        ),
        mesh=_MESH,
        compiler_params=_PARAMS,
        scratch_types=[
            pltpu.VMEM((NW * L,), jnp.float32),
            pltpu.VMEM((NW * L,), jnp.float32),
            pltpu.VMEM((L,), jnp.int32),
            pltpu.VMEM((1, D), jnp.float32),
            pltpu.VMEM((D,), jnp.float32),
            pltpu.SemaphoreType.DMA,
        ],
    )(codebook, out_d, out_i)

    return row, idx[0], dist[0]


# speedup vs baseline: 1.4859x; 1.4859x over previous
import functools
import jax
import jax.numpy as jnp
from jax import lax
from jax.experimental import pallas as pl
from jax.experimental.pallas import tpu as pltpu, tpu_sc as plsc

D = 256
N = 8192
NC = 2
NS = 16
NW = NC * NS
RPW = N // NW
L = 16

_MESH = plsc.VectorSubcoreMesh(
    core_axis_name="c", subcore_axis_name="s", num_cores=NC, num_subcores=NS)
_PARAMS = pltpu.CompilerParams(needs_layout_passes=False)


def _stage1(cb_hbm, z_hbm, out_d, out_i, cb_v, z_v, res_v):
    c = lax.axis_index("c")
    s = lax.axis_index("s")
    wid = c * NS + s
    base_row = wid * RPW

    pltpu.sync_copy(cb_hbm.at[pl.ds(base_row, RPW)], cb_v)
    pltpu.sync_copy(z_hbm, z_v)

    z_vecs = [z_v[0, pl.ds(L * d, L)] for d in range(D // L)]

    R = 4

    def group_body(g, carry):
        best, bidx = carry
        sums = []
        for j in range(R):
            acc = jnp.zeros((L,), jnp.float32)
            for d in range(D // L):
                t = cb_v[g * R + j, pl.ds(L * d, L)] - z_vecs[d]
                acc = acc + t * t
            sums.append(jnp.sum(acc))
        for j in range(R):
            r = g * R + j
            m = sums[j] < best
            best = jnp.where(m, sums[j], best)
            bidx = jnp.where(m, base_row + r, bidx)
        return best, bidx

    init = (jnp.float32(jnp.inf), jnp.int32(0))
    best, bidx = lax.fori_loop(0, RPW // R, group_body, init)

    res_v[pl.ds(0, L)] = jnp.broadcast_to(best, (L,))
    res_v[pl.ds(L, L)] = plsc.bitcast(
        jnp.broadcast_to(bidx, (L,)).astype(jnp.int32), jnp.float32)
    pltpu.sync_copy(res_v.at[pl.ds(0, L)], out_d.at[pl.ds(wid * L, L)])
    pltpu.sync_copy(res_v.at[pl.ds(L, L)], out_i.at[pl.ds(wid * L, L)])


def _stage2(cb2d, out_d, out_i, row_out, idx_out, dist_out,
            d_v, i_v, iv_v, row_v, o_v, sem):
    c = lax.axis_index("c")
    s = lax.axis_index("s")
    wid = c * NS + s

    @pl.when(wid == 0)
    def _():
        pltpu.sync_copy(out_d, d_v)
        pltpu.sync_copy(out_i, i_v)
        best = jnp.full((L,), jnp.inf, jnp.float32)
        bidx = jnp.zeros((L,), jnp.int32)
        for w in range(NW):
            dw = d_v[pl.ds(w * L, L)]
            iw = plsc.bitcast(i_v[pl.ds(w * L, L)], jnp.int32)
            m = dw < best
            best = jnp.where(m, dw, best)
            bidx = jnp.where(m, iw, bidx)
        iv_v[...] = bidx
        pltpu.async_copy(cb2d.at[iv_v.at[pl.ds(0, 1)]], row_v, sem).wait()
        for d in range(D // L):
            o_v[pl.ds(L * d, L)] = row_v[0, pl.ds(L * d, L)]
        pltpu.sync_copy(o_v, row_out)
        pltpu.sync_copy(iv_v, idx_out)
        bi = plsc.bitcast(best, jnp.int32)
        g = plsc.bitcast(
            jnp.int32(0x1FBD1DF5) + lax.shift_right_logical(bi, 1),
            jnp.float32)
        half = jnp.float32(0.5)
        for _ in range(4):
            g = half * (g + best / g)
        g = jnp.where(best == 0.0, jnp.zeros((L,), jnp.float32), g)
        d_v[pl.ds(0, L)] = g
        pltpu.sync_copy(d_v.at[pl.ds(0, L)], dist_out)


@jax.jit
def kernel(z, codebook):
    out_d, out_i = pl.kernel(
        _stage1,
        out_type=(
            jax.ShapeDtypeStruct((NW * L,), jnp.float32),
            jax.ShapeDtypeStruct((NW * L,), jnp.float32),
        ),
        mesh=_MESH,
        compiler_params=_PARAMS,
        scratch_types=[
            pltpu.VMEM((RPW, D), jnp.float32),
            pltpu.VMEM((1, D), jnp.float32),
            pltpu.VMEM((2 * L,), jnp.float32),
        ],
    )(codebook, z)

    row, idx, dist = pl.kernel(
        _stage2,
        out_type=(
            jax.ShapeDtypeStruct((D,), jnp.float32),
            jax.ShapeDtypeStruct((L,), jnp.int32),
            jax.ShapeDtypeStruct((L,), jnp.float32),
        ),
        mesh=_MESH,
        compiler_params=_PARAMS,
        scratch_types=[
            pltpu.VMEM((NW * L,), jnp.float32),
            pltpu.VMEM((NW * L,), jnp.float32),
            pltpu.VMEM((L,), jnp.int32),
            pltpu.VMEM((1, D), jnp.float32),
            pltpu.VMEM((D,), jnp.float32),
            pltpu.SemaphoreType.DMA,
        ],
    )(codebook, out_d, out_i)

    return row, idx[0], dist[0]


# speedup vs baseline: 1.6396x; 1.1034x over previous
import jax
import jax.numpy as jnp
from jax import lax
from jax.experimental import pallas as pl
from jax.experimental.pallas import tpu as pltpu, tpu_sc as plsc

D = 256
N = 8192
NC = 2
NS = 16
NW = NC * NS
RPW = N // NW
L = 16
NQ = 4
CR = RPW // NQ

_MESH = plsc.VectorSubcoreMesh(
    core_axis_name="c", subcore_axis_name="s", num_cores=NC, num_subcores=NS)
_PARAMS = pltpu.CompilerParams(needs_layout_passes=False)


def _stage1(cb_hbm, z_hbm, out_d, out_i, cb_v, z_v, res_v, resi_v, sems):
    c = lax.axis_index("c")
    s = lax.axis_index("s")
    wid = c * NS + s
    base_row = wid * RPW

    copies = [
        pltpu.async_copy(
            cb_hbm.at[pl.ds(base_row + q * CR, CR)], cb_v.at[q], sems.at[q])
        for q in range(NQ)
    ]
    pltpu.sync_copy(z_hbm, z_v)
    z_vecs = [z_v[0, pl.ds(L * d, L)] for d in range(D // L)]

    R = 4

    best = jnp.float32(jnp.inf)
    bidx = jnp.int32(0)
    for q in range(NQ):
        copies[q].wait()

        def group_body(g, carry, q=q):
            best, bidx = carry
            sums = []
            for j in range(R):
                acc = jnp.zeros((L,), jnp.float32)
                for d in range(D // L):
                    t = cb_v[q, g * R + j, pl.ds(L * d, L)] - z_vecs[d]
                    acc = acc + t * t
                sums.append(jnp.sum(acc))
            for j in range(R):
                r = q * CR + g * R + j
                m = sums[j] < best
                best = jnp.where(m, sums[j], best)
                bidx = jnp.where(m, base_row + r, bidx)
            return best, bidx

        best, bidx = lax.fori_loop(0, CR // R, group_body, (best, bidx))

    res_v[...] = jnp.broadcast_to(best, (L,))
    resi_v[...] = jnp.broadcast_to(bidx, (L,)).astype(jnp.int32)
    pltpu.sync_copy(res_v, out_d.at[pl.ds(wid * L, L)])
    pltpu.sync_copy(resi_v, out_i.at[pl.ds(wid * L, L)])


def _merge(d_ref, i_ref, cb_any, row_ref, idx_ref, dist_ref, rowbuf, sem):
    d = d_ref[...]
    i = i_ref[...]
    dmin = jnp.min(d)
    idx = jnp.min(jnp.where(d == dmin, i, jnp.int32(N)))
    cp = pltpu.make_async_copy(cb_any.at[pl.ds(idx, 1)], rowbuf, sem)
    cp.start()
    idx_ref[0, 0] = idx
    dist_ref[0, 0] = jnp.sqrt(dmin)
    cp.wait()
    row_ref[...] = rowbuf[...]


@jax.jit
def kernel(z, codebook):
    out_d, out_i = pl.kernel(
        _stage1,
        out_type=(
            jax.ShapeDtypeStruct((NW * L,), jnp.float32),
            jax.ShapeDtypeStruct((NW * L,), jnp.int32),
        ),
        mesh=_MESH,
        compiler_params=_PARAMS,
        scratch_types=[
            pltpu.VMEM((NQ, CR, D), jnp.float32),
            pltpu.VMEM((1, D), jnp.float32),
            pltpu.VMEM((L,), jnp.float32),
            pltpu.VMEM((L,), jnp.int32),
            pltpu.SemaphoreType.DMA((NQ,)),
        ],
    )(codebook, z)

    row, idx, dist = pl.pallas_call(
        _merge,
        out_shape=(
            jax.ShapeDtypeStruct((1, D), jnp.float32),
            jax.ShapeDtypeStruct((1, 1), jnp.int32),
            jax.ShapeDtypeStruct((1, 1), jnp.float32),
        ),
        in_specs=[
            pl.BlockSpec(memory_space=pltpu.VMEM),
            pl.BlockSpec(memory_space=pltpu.VMEM),
            pl.BlockSpec(memory_space=pl.ANY),
        ],
        out_specs=(
            pl.BlockSpec(memory_space=pltpu.VMEM),
            pl.BlockSpec(memory_space=pltpu.SMEM),
            pl.BlockSpec(memory_space=pltpu.SMEM),
        ),
        scratch_shapes=[
            pltpu.VMEM((1, D), jnp.float32),
            pltpu.SemaphoreType.DMA,
        ],
    )(out_d, out_i, codebook)

    return row[0], idx[0, 0], dist[0, 0]
